# Initial kernel scaffold; baseline (speedup 1.0000x reference)
import jax
import jax.numpy as jnp
from jax.experimental import pallas as pl


def kernel(x, edge_index, batch, conv1_w, bn1_g, bn1_b, gcn1_w, gcn1_b, gcn2_w, gcn2_b, conv2_w, bn2_g, bn2_b, conv3_w, bn3_g, bn3_b, lin_w, lin_b):
    raise NotImplementedError("write your pallas kernel here")


# speedup vs baseline: 15.6521x; 15.6521x over previous
import jax
import jax.numpy as jnp
from jax import lax
from jax.experimental import pallas as pl
from jax.experimental.pallas import tpu as pltpu
from jax.experimental.pallas import tpu_sc as plsc

N = 10000
NP = 10240
T = 16
E = 160000
K0 = 16
D = 32
B = 100
BP = 104
C = 4
WC = 128
NS = 16
NC = 2
KBLK = 128
ET = 163840
EPT = ET // NS
EPW = ET // (NS * NC)
RW = NP // NS
NB = 2000


def _deg_body(dstp, ones_h, zeros_h, hist, acc, idx_d, ones_v):
    c = lax.axis_index("c")
    s = lax.axis_index("s")
    w = c * NS + s
    pltpu.sync_copy(zeros_h, acc.at[pl.ds(s * RW, RW)])
    pltpu.sync_copy(ones_h, ones_v)
    plsc.subcore_barrier()

    def blk(b, carry):
        off = w * EPW + b * KBLK
        pltpu.sync_copy(dstp.at[pl.ds(off, KBLK)], idx_d)
        pltpu.sync_copy(ones_v, acc.at[idx_d], add=True)
        return carry

    lax.fori_loop(0, EPW // KBLK, blk, 0)
    plsc.subcore_barrier()
    pltpu.sync_copy(acc.at[pl.ds(s * RW, RW)],
                    hist.at[pl.ds(c * NP + s * RW, RW)])


_deg = pl.kernel(
    _deg_body,
    out_type=jax.ShapeDtypeStruct((NC * NP, WC), jnp.float32),
    mesh=plsc.VectorSubcoreMesh(core_axis_name="c", subcore_axis_name="s",
                                num_cores=NC, num_subcores=NS),
    scratch_types=[
        pltpu.VMEM_SHARED((NP, WC), jnp.float32),
        pltpu.VMEM((KBLK,), jnp.int32),
        pltpu.VMEM((KBLK, WC), jnp.float32),
    ],
)


def _spmm_body(gtab, srcp, dstp, out, acc, idx_s, idx_d, msg, sem):
    c = lax.axis_index("c")
    s = lax.axis_index("s")
    row0 = s * RW
    for r in range(2):
        q = 2 * c + r
        pltpu.sync_copy(gtab.at[pl.ds(q * NP + row0, RW)],
                        acc.at[pl.ds(row0, RW)])
        plsc.subcore_barrier()
        base = q * NP

        def blk(b, carry):
            off = s * EPT + b * KBLK
            pltpu.sync_copy(srcp.at[pl.ds(off, KBLK)], idx_s)
            pltpu.sync_copy(dstp.at[pl.ds(off, KBLK)], idx_d)
            for j in range(KBLK // 16):
                idx_s[pl.ds(j * 16, 16)] = idx_s[pl.ds(j * 16, 16)] + base
            pltpu.async_copy(gtab.at[idx_s], msg, sem).wait()
            pltpu.sync_copy(msg, acc.at[idx_d], add=True)
            return carry

        lax.fori_loop(0, EPT // KBLK, blk, 0)
        plsc.subcore_barrier()
        pltpu.sync_copy(acc.at[pl.ds(row0, RW)],
                        out.at[pl.ds(q * NP + row0, RW)])


_spmm = pl.kernel(
    _spmm_body,
    out_type=jax.ShapeDtypeStruct((C * NP, WC), jnp.float32),
    mesh=plsc.VectorSubcoreMesh(core_axis_name="c", subcore_axis_name="s",
                                num_cores=NC, num_subcores=NS),
    scratch_types=[
        pltpu.VMEM_SHARED((NP, WC), jnp.float32),
        pltpu.VMEM((KBLK,), jnp.int32),
        pltpu.VMEM((KBLK,), jnp.int32),
        pltpu.VMEM((KBLK, WC), jnp.float32),
        pltpu.SemaphoreType.DMA,
    ],
)


def _dinv_col(hist_ref):
    hist = hist_ref[...]
    return lax.rsqrt(1.0 + hist[0] + hist[1])[:, 0:1]


def _tca_body(x_ref, hist_ref, w1_ref, sc1_ref, b1_ref, w1g_ref, out_ref):
    dcol = _dinv_col(hist_ref)
    x = x_ref[...]
    parts = [[] for _ in range(C)]
    for t in range(T):
        h = None
        for j in range(7):
            tt = t + j - 3
            if 0 <= tt < T:
                term = x[:, tt:tt + 1] * w1_ref[j:j + 1, :]
                h = term if h is None else h + term
        h = jnp.maximum(h * sc1_ref[...] + b1_ref[...], 0.0)
        m = jnp.dot(h, w1g_ref[...], preferred_element_type=jnp.float32)
        parts[t // 4].append(m * dcol)
    for q in range(C):
        out_ref[q] = jnp.concatenate(parts[q], axis=1)


_tca = pl.pallas_call(
    _tca_body,
    grid=(N // NB,),
    in_specs=[
        pl.BlockSpec((NB, T), lambda i: (i, 0)),
        pl.BlockSpec((2, NB, WC), lambda i: (0, i, 0)),
        pl.BlockSpec((7, K0), lambda i: (0, 0)),
        pl.BlockSpec((1, K0), lambda i: (0, 0)),
        pl.BlockSpec((1, K0), lambda i: (0, 0)),
        pl.BlockSpec((K0, D), lambda i: (0, 0)),
    ],
    out_specs=pl.BlockSpec((C, NB, WC), lambda i: (0, i, 0)),
    out_shape=jax.ShapeDtypeStruct((C, NP, WC), jnp.float32),
)


def _tcb_body(u_ref, hist_ref, b1t_ref, w2_ref, out_ref):
    dcol = _dinv_col(hist_ref)
    for q in range(C):
        z = jnp.maximum(u_ref[q] * dcol + b1t_ref[...], 0.0)
        ms = [jnp.dot(z[:, 32 * r:32 * r + 32], w2_ref[...],
                      preferred_element_type=jnp.float32) for r in range(4)]
        out_ref[q] = jnp.concatenate(ms, axis=1) * dcol


_tcb = pl.pallas_call(
    _tcb_body,
    grid=(N // NB,),
    in_specs=[
        pl.BlockSpec((C, NB, WC), lambda i: (0, i, 0)),
        pl.BlockSpec((2, NB, WC), lambda i: (0, i, 0)),
        pl.BlockSpec((1, WC), lambda i: (0, 0)),
        pl.BlockSpec((D, D), lambda i: (0, 0)),
    ],
    out_specs=pl.BlockSpec((C, NB, WC), lambda i: (0, i, 0)),
    out_shape=jax.ShapeDtypeStruct((C, NP, WC), jnp.float32),
)


def _tcc_body(u_ref, hist_ref, b2t_ref, w2c_ref, sc2_ref, bb2_ref,
              batch_ref, out_ref):
    i = pl.program_id(0)
    dcol = _dinv_col(hist_ref)
    g = jnp.concatenate(
        [jnp.maximum(u_ref[q] * dcol + b2t_ref[...], 0.0) for q in range(C)],
        axis=1)
    vs = []
    for t in range(T - 4):
        a = None
        for j in range(5):
            col = 32 * (t + j)
            term = jnp.dot(g[:, col:col + 32], w2c_ref[32 * j:32 * j + 32, :],
                           preferred_element_type=jnp.float32)
            a = term if a is None else a + term
        vs.append(a * sc2_ref[...] + bb2_ref[...])
    us = [jnp.maximum((vs[2 * t] + vs[2 * t + 1]) * 0.5, 0.0)
          for t in range(6)]
    onecol = (lax.broadcasted_iota(jnp.int32, (NB, 64), 1) == 0)
    uall = jnp.concatenate(us + [onecol.astype(jnp.float32)], axis=1)
    oh = (lax.broadcasted_iota(jnp.int32, (BP, NB), 0) == batch_ref[0])
    pp = jnp.dot(oh.astype(jnp.float32), uall,
                 preferred_element_type=jnp.float32)

    @pl.when(i == 0)
    def _():
        out_ref[...] = pp

    @pl.when(i > 0)
    def _():
        out_ref[...] += pp


_tcc = pl.pallas_call(
    _tcc_body,
    grid=(N // NB,),
    in_specs=[
        pl.BlockSpec((C, NB, WC), lambda i: (0, i, 0)),
        pl.BlockSpec((2, NB, WC), lambda i: (0, i, 0)),
        pl.BlockSpec((1, WC), lambda i: (0, 0)),
        pl.BlockSpec((5 * D, D), lambda i: (0, 0)),
        pl.BlockSpec((1, D), lambda i: (0, 0)),
        pl.BlockSpec((1, D), lambda i: (0, 0)),
        pl.BlockSpec((1, 1, NB), lambda i: (i, 0, 0)),
    ],
    out_specs=pl.BlockSpec((BP, 256), lambda i: (0, 0)),
    out_shape=jax.ShapeDtypeStruct((BP, 256), jnp.float32),
)


def _tcd_body(p_ref, w3_ref, sc3_ref, b3_ref, lw_ref, lb_ref, out_ref):
    pe = p_ref[...]
    cnt = jnp.maximum(pe[:, 192:193], 1.0)
    p = pe[:, 0:192] / cnt
    vs = []
    for t in range(5):
        a = (jnp.dot(p[:, 32 * t:32 * t + 32], w3_ref[0:32, :],
                     preferred_element_type=jnp.float32) +
             jnp.dot(p[:, 32 * t + 32:32 * t + 64], w3_ref[32:64, :],
                     preferred_element_type=jnp.float32))
        vs.append(a * sc3_ref[...] + b3_ref[...])
    f0 = jnp.maximum((vs[0] + vs[1]) * 0.5, 0.0)
    f1 = jnp.maximum((vs[2] + vs[3]) * 0.5, 0.0)
    flat = jnp.concatenate([f0, f1], axis=1)
    logits = jnp.dot(flat, lw_ref[...],
                     preferred_element_type=jnp.float32) + lb_ref[...]
    mx = jnp.max(logits, axis=1, keepdims=True)
    e = jnp.exp(logits - mx)
    lse = jnp.log(jnp.sum(e, axis=1, keepdims=True)) + mx
    out_ref[...] = logits - lse


_tcd = pl.pallas_call(
    _tcd_body,
    in_specs=[
        pl.BlockSpec((BP, 256), lambda: (0, 0)),
        pl.BlockSpec((64, D), lambda: (0, 0)),
        pl.BlockSpec((1, D), lambda: (0, 0)),
        pl.BlockSpec((1, D), lambda: (0, 0)),
        pl.BlockSpec((64, 128), lambda: (0, 0)),
        pl.BlockSpec((1, 128), lambda: (0, 0)),
    ],
    out_specs=pl.BlockSpec((BP, 128), lambda: (0, 0)),
    out_shape=jax.ShapeDtypeStruct((BP, 128), jnp.float32),
)


def kernel(x, edge_index, batch, conv1_w, bn1_g, bn1_b, gcn1_w, gcn1_b,
           gcn2_w, gcn2_b, conv2_w, bn2_g, bn2_b, conv3_w, bn3_g, bn3_b,
           lin_w, lin_b):
    f32 = jnp.float32
    pad = ET - E
    srcp = jnp.concatenate([edge_index[0],
                            jnp.zeros((pad,), jnp.int32)])
    dstp = jnp.concatenate([edge_index[1],
                            jnp.full((pad,), N, jnp.int32)])
    ones_h = jnp.ones((KBLK, WC), f32)
    zeros_h = jnp.zeros((RW, WC), f32)
    hist = _deg(dstp, ones_h, zeros_h).reshape(2, NP, WC)

    rs = 1.0 / jnp.sqrt(jnp.asarray(1.0 + 1e-5, f32))
    w1 = conv1_w[:, 0, 0, :]
    sc1 = (bn1_g * rs).reshape(1, K0)
    b1 = bn1_b.reshape(1, K0)
    g1 = _tca(x, hist, w1, sc1, b1, gcn1_w)

    def _jnp_spmm(gtab, srcp, dstp):
        out = gtab
        for q in range(C):
            seg = gtab[q * NP:(q + 1) * NP]
            acc = seg.at[dstp].add(seg[srcp])
            out = out.at[q * NP:(q + 1) * NP].set(acc)
        return out

    u1 = _spmm(g1.reshape(C * NP, WC), srcp, dstp).reshape(C, NP, WC)
    b1t = jnp.tile(gcn1_b, C).reshape(1, WC)
    g2 = _tcb(u1, hist, b1t, gcn2_w)

    u2 = _spmm(g2.reshape(C * NP, WC), srcp, dstp).reshape(C, NP, WC)
    b2t = jnp.tile(gcn2_b, C).reshape(1, WC)
    w2c = conv2_w[:, 0, :, :].reshape(5 * D, D)
    sc2 = (bn2_g * rs).reshape(1, D)
    bb2 = bn2_b.reshape(1, D)
    pext = _tcc(u2, hist, b2t, w2c, sc2, bb2, batch.reshape(N // NB, 1, NB))

    w3 = conv3_w[:, 0, :, :].reshape(2 * D, D)
    sc3 = (bn3_g * rs).reshape(1, D)
    b3 = bn3_b.reshape(1, D)
    lw = jnp.concatenate([lin_w, jnp.zeros((2 * D, 124), f32)], axis=1)
    lb = jnp.concatenate([lin_b, jnp.full((124,), -1e30, f32)]).reshape(1, 128)
    out = _tcd(pext, w3, sc3, b3, lw, lb)
    return out[:B, :4]


# speedup vs baseline: 21.3618x; 1.3648x over previous
import jax
import jax.numpy as jnp
from jax import lax
from jax.experimental import pallas as pl
from jax.experimental.pallas import tpu as pltpu
from jax.experimental.pallas import tpu_sc as plsc

N = 10000
NP = 10240
T = 16
E = 160000
K0 = 16
D = 32
B = 100
BP = 104
C = 4
WC = 128
NS = 16
NC = 2
KBLK = 128
ET = 163840
EPT = ET // NS
EPW = ET // (NS * NC)
RW = NP // NS
NB = 2000


def _deg_body(dstp, ones_h, zeros_h, hist, acc, idx_d, ones_v):
    c = lax.axis_index("c")
    s = lax.axis_index("s")
    w = c * NS + s
    pltpu.sync_copy(zeros_h, acc.at[pl.ds(s * RW, RW)])
    pltpu.sync_copy(ones_h, ones_v)
    plsc.subcore_barrier()

    def blk(b, carry):
        off = w * EPW + b * KBLK
        pltpu.sync_copy(dstp.at[pl.ds(off, KBLK)], idx_d)
        pltpu.sync_copy(ones_v, acc.at[idx_d], add=True)
        return carry

    lax.fori_loop(0, EPW // KBLK, blk, 0)
    plsc.subcore_barrier()
    pltpu.sync_copy(acc.at[pl.ds(s * RW, RW)],
                    hist.at[pl.ds(c * NP + s * RW, RW)])


_deg = pl.kernel(
    _deg_body,
    out_type=jax.ShapeDtypeStruct((NC * NP, WC), jnp.float32),
    mesh=plsc.VectorSubcoreMesh(core_axis_name="c", subcore_axis_name="s",
                                num_cores=NC, num_subcores=NS),
    scratch_types=[
        pltpu.VMEM_SHARED((NP, WC), jnp.float32),
        pltpu.VMEM((KBLK,), jnp.int32),
        pltpu.VMEM((KBLK, WC), jnp.float32),
    ],
)


NBLK = EPT // KBLK
GB = 16


def _spmm_body(gtab, src2h, dst2h, out, acc, srcoff, dst2,
               msga, msgb, gsa, gsb):
    c = lax.axis_index("c")
    s = lax.axis_index("s")
    row0 = s * RW
    for r in range(2):
        q = 2 * c + r
        base = q * NP
        pltpu.sync_copy(gtab.at[pl.ds(base + row0, RW)],
                        acc.at[pl.ds(row0, RW)])
        plsc.subcore_barrier()

        def group(gi, carry):
            g0 = s * NBLK + gi * GB
            pltpu.sync_copy(src2h.at[pl.ds(g0, GB)], srcoff)
            pltpu.sync_copy(dst2h.at[pl.ds(g0, GB)], dst2)
            for i in range(GB):
                for j in range(KBLK // 16):
                    srcoff[i, pl.ds(16 * j, 16)] = (
                        srcoff[i, pl.ds(16 * j, 16)] + base)
            pltpu.async_copy(gtab.at[srcoff.at[0]], msga, gsa)
            for k in range(GB):
                cur, nxt = (msga, msgb) if k % 2 == 0 else (msgb, msga)
                csem, nsem = (gsa, gsb) if k % 2 == 0 else (gsb, gsa)
                if k + 1 < GB:
                    pltpu.async_copy(gtab.at[srcoff.at[k + 1]], nxt, nsem)
                pltpu.make_async_copy(gtab.at[srcoff.at[k]], cur, csem).wait()
                pltpu.sync_copy(cur, acc.at[dst2.at[k]], add=True)
            return carry

        lax.fori_loop(0, NBLK // GB, group, 0)
        plsc.subcore_barrier()
        pltpu.sync_copy(acc.at[pl.ds(row0, RW)],
                        out.at[pl.ds(q * NP + row0, RW)])


_spmm = pl.kernel(
    _spmm_body,
    out_type=jax.ShapeDtypeStruct((C * NP, WC), jnp.float32),
    mesh=plsc.VectorSubcoreMesh(core_axis_name="c", subcore_axis_name="s",
                                num_cores=NC, num_subcores=NS),
    scratch_types=[
        pltpu.VMEM_SHARED((NP, WC), jnp.float32),
        pltpu.VMEM((GB, KBLK), jnp.int32),
        pltpu.VMEM((GB, KBLK), jnp.int32),
        pltpu.VMEM((KBLK, WC), jnp.float32),
        pltpu.VMEM((KBLK, WC), jnp.float32),
        pltpu.SemaphoreType.DMA,
        pltpu.SemaphoreType.DMA,
    ],
)


def _dinv_col(hist_ref):
    hist = hist_ref[...]
    return lax.rsqrt(1.0 + hist[0] + hist[1])[:, 0:1]


def _tca_body(x_ref, hist_ref, w1_ref, sc1_ref, b1_ref, w1g_ref, out_ref):
    dcol = _dinv_col(hist_ref)
    x = x_ref[...]
    parts = [[] for _ in range(C)]
    for t in range(T):
        h = None
        for j in range(7):
            tt = t + j - 3
            if 0 <= tt < T:
                term = x[:, tt:tt + 1] * w1_ref[j:j + 1, :]
                h = term if h is None else h + term
        h = jnp.maximum(h * sc1_ref[...] + b1_ref[...], 0.0)
        m = jnp.dot(h, w1g_ref[...], preferred_element_type=jnp.float32)
        parts[t // 4].append(m * dcol)
    for q in range(C):
        out_ref[q] = jnp.concatenate(parts[q], axis=1)


_tca = pl.pallas_call(
    _tca_body,
    grid=(N // NB,),
    in_specs=[
        pl.BlockSpec((NB, T), lambda i: (i, 0)),
        pl.BlockSpec((2, NB, WC), lambda i: (0, i, 0)),
        pl.BlockSpec((7, K0), lambda i: (0, 0)),
        pl.BlockSpec((1, K0), lambda i: (0, 0)),
        pl.BlockSpec((1, K0), lambda i: (0, 0)),
        pl.BlockSpec((K0, D), lambda i: (0, 0)),
    ],
    out_specs=pl.BlockSpec((C, NB, WC), lambda i: (0, i, 0)),
    out_shape=jax.ShapeDtypeStruct((C, NP, WC), jnp.float32),
)


def _tcb_body(u_ref, hist_ref, b1t_ref, w2_ref, out_ref):
    dcol = _dinv_col(hist_ref)
    for q in range(C):
        z = jnp.maximum(u_ref[q] * dcol + b1t_ref[...], 0.0)
        ms = [jnp.dot(z[:, 32 * r:32 * r + 32], w2_ref[...],
                      preferred_element_type=jnp.float32) for r in range(4)]
        out_ref[q] = jnp.concatenate(ms, axis=1) * dcol


_tcb = pl.pallas_call(
    _tcb_body,
    grid=(N // NB,),
    in_specs=[
        pl.BlockSpec((C, NB, WC), lambda i: (0, i, 0)),
        pl.BlockSpec((2, NB, WC), lambda i: (0, i, 0)),
        pl.BlockSpec((1, WC), lambda i: (0, 0)),
        pl.BlockSpec((D, D), lambda i: (0, 0)),
    ],
    out_specs=pl.BlockSpec((C, NB, WC), lambda i: (0, i, 0)),
    out_shape=jax.ShapeDtypeStruct((C, NP, WC), jnp.float32),
)


def _tcc_body(u_ref, hist_ref, b2t_ref, w2c_ref, sc2_ref, bb2_ref,
              batch_ref, out_ref):
    i = pl.program_id(0)
    dcol = _dinv_col(hist_ref)
    g = jnp.concatenate(
        [jnp.maximum(u_ref[q] * dcol + b2t_ref[...], 0.0) for q in range(C)],
        axis=1)
    vs = []
    for t in range(T - 4):
        a = None
        for j in range(5):
            col = 32 * (t + j)
            term = jnp.dot(g[:, col:col + 32], w2c_ref[32 * j:32 * j + 32, :],
                           preferred_element_type=jnp.float32)
            a = term if a is None else a + term
        vs.append(a * sc2_ref[...] + bb2_ref[...])
    us = [jnp.maximum((vs[2 * t] + vs[2 * t + 1]) * 0.5, 0.0)
          for t in range(6)]
    onecol = (lax.broadcasted_iota(jnp.int32, (NB, 64), 1) == 0)
    uall = jnp.concatenate(us + [onecol.astype(jnp.float32)], axis=1)
    oh = (lax.broadcasted_iota(jnp.int32, (BP, NB), 0) == batch_ref[0])
    pp = jnp.dot(oh.astype(jnp.float32), uall,
                 preferred_element_type=jnp.float32)

    @pl.when(i == 0)
    def _():
        out_ref[...] = pp

    @pl.when(i > 0)
    def _():
        out_ref[...] += pp


_tcc = pl.pallas_call(
    _tcc_body,
    grid=(N // NB,),
    in_specs=[
        pl.BlockSpec((C, NB, WC), lambda i: (0, i, 0)),
        pl.BlockSpec((2, NB, WC), lambda i: (0, i, 0)),
        pl.BlockSpec((1, WC), lambda i: (0, 0)),
        pl.BlockSpec((5 * D, D), lambda i: (0, 0)),
        pl.BlockSpec((1, D), lambda i: (0, 0)),
        pl.BlockSpec((1, D), lambda i: (0, 0)),
        pl.BlockSpec((1, 1, NB), lambda i: (i, 0, 0)),
    ],
    out_specs=pl.BlockSpec((BP, 256), lambda i: (0, 0)),
    out_shape=jax.ShapeDtypeStruct((BP, 256), jnp.float32),
)


def _tcd_body(p_ref, w3_ref, sc3_ref, b3_ref, lw_ref, lb_ref, out_ref):
    pe = p_ref[...]
    cnt = jnp.maximum(pe[:, 192:193], 1.0)
    p = pe[:, 0:192] / cnt
    vs = []
    for t in range(5):
        a = (jnp.dot(p[:, 32 * t:32 * t + 32], w3_ref[0:32, :],
                     preferred_element_type=jnp.float32) +
             jnp.dot(p[:, 32 * t + 32:32 * t + 64], w3_ref[32:64, :],
                     preferred_element_type=jnp.float32))
        vs.append(a * sc3_ref[...] + b3_ref[...])
    f0 = jnp.maximum((vs[0] + vs[1]) * 0.5, 0.0)
    f1 = jnp.maximum((vs[2] + vs[3]) * 0.5, 0.0)
    flat = jnp.concatenate([f0, f1], axis=1)
    logits = jnp.dot(flat, lw_ref[...],
                     preferred_element_type=jnp.float32) + lb_ref[...]
    mx = jnp.max(logits, axis=1, keepdims=True)
    e = jnp.exp(logits - mx)
    lse = jnp.log(jnp.sum(e, axis=1, keepdims=True)) + mx
    out_ref[...] = logits - lse


_tcd = pl.pallas_call(
    _tcd_body,
    in_specs=[
        pl.BlockSpec((BP, 256), lambda: (0, 0)),
        pl.BlockSpec((64, D), lambda: (0, 0)),
        pl.BlockSpec((1, D), lambda: (0, 0)),
        pl.BlockSpec((1, D), lambda: (0, 0)),
        pl.BlockSpec((64, 128), lambda: (0, 0)),
        pl.BlockSpec((1, 128), lambda: (0, 0)),
    ],
    out_specs=pl.BlockSpec((BP, 128), lambda: (0, 0)),
    out_shape=jax.ShapeDtypeStruct((BP, 128), jnp.float32),
)


def kernel(x, edge_index, batch, conv1_w, bn1_g, bn1_b, gcn1_w, gcn1_b,
           gcn2_w, gcn2_b, conv2_w, bn2_g, bn2_b, conv3_w, bn3_g, bn3_b,
           lin_w, lin_b):
    f32 = jnp.float32
    pad = ET - E
    srcp = jnp.concatenate([edge_index[0],
                            jnp.zeros((pad,), jnp.int32)])
    dstp = jnp.concatenate([edge_index[1],
                            jnp.full((pad,), N, jnp.int32)])
    ones_h = jnp.ones((KBLK, WC), f32)
    zeros_h = jnp.zeros((RW, WC), f32)
    hist = _deg(dstp, ones_h, zeros_h).reshape(2, NP, WC)

    rs = 1.0 / jnp.sqrt(jnp.asarray(1.0 + 1e-5, f32))
    w1 = conv1_w[:, 0, 0, :]
    sc1 = (bn1_g * rs).reshape(1, K0)
    b1 = bn1_b.reshape(1, K0)
    g1 = _tca(x, hist, w1, sc1, b1, gcn1_w)

    def _jnp_spmm(gtab, srcp, dstp):
        out = gtab
        for q in range(C):
            seg = gtab[q * NP:(q + 1) * NP]
            acc = seg.at[dstp].add(seg[srcp])
            out = out.at[q * NP:(q + 1) * NP].set(acc)
        return out

    src2h = srcp.reshape(ET // KBLK, KBLK)
    dst2h = dstp.reshape(ET // KBLK, KBLK)
    u1 = _spmm(g1.reshape(C * NP, WC), src2h, dst2h).reshape(C, NP, WC)
    b1t = jnp.tile(gcn1_b, C).reshape(1, WC)
    g2 = _tcb(u1, hist, b1t, gcn2_w)

    u2 = _spmm(g2.reshape(C * NP, WC), src2h, dst2h).reshape(C, NP, WC)
    b2t = jnp.tile(gcn2_b, C).reshape(1, WC)
    w2c = conv2_w[:, 0, :, :].reshape(5 * D, D)
    sc2 = (bn2_g * rs).reshape(1, D)
    bb2 = bn2_b.reshape(1, D)
    pext = _tcc(u2, hist, b2t, w2c, sc2, bb2, batch.reshape(N // NB, 1, NB))

    w3 = conv3_w[:, 0, :, :].reshape(2 * D, D)
    sc3 = (bn3_g * rs).reshape(1, D)
    b3 = bn3_b.reshape(1, D)
    lw = jnp.concatenate([lin_w, jnp.zeros((2 * D, 124), f32)], axis=1)
    lb = jnp.concatenate([lin_b, jnp.full((124,), -1e30, f32)]).reshape(1, 128)
    out = _tcd(pext, w3, sc3, b3, lw, lb)
    return out[:B, :4]


# speedup vs baseline: 21.3651x; 1.0002x over previous
import jax
import jax.numpy as jnp
from jax import lax
from jax.experimental import pallas as pl
from jax.experimental.pallas import tpu as pltpu
from jax.experimental.pallas import tpu_sc as plsc

N = 10000
NP = 10240
T = 16
E = 160000
K0 = 16
D = 32
B = 100
BP = 104
C = 4
WC = 128
NS = 16
NC = 2
KBLK = 128
ET = 163840
EPT = ET // NS
EPW = ET // (NS * NC)
RW = NP // NS
NB = 2000


def _deg_body(dstp, ones_h, zeros_h, hist, acc, idx_d, ones_v):
    c = lax.axis_index("c")
    s = lax.axis_index("s")
    w = c * NS + s
    pltpu.sync_copy(zeros_h, acc.at[pl.ds(s * RW, RW)])
    pltpu.sync_copy(ones_h, ones_v)
    plsc.subcore_barrier()

    def blk(b, carry):
        off = w * EPW + b * KBLK
        pltpu.sync_copy(dstp.at[pl.ds(off, KBLK)], idx_d)
        pltpu.sync_copy(ones_v, acc.at[idx_d], add=True)
        return carry

    lax.fori_loop(0, EPW // KBLK, blk, 0)
    plsc.subcore_barrier()
    pltpu.sync_copy(acc.at[pl.ds(s * RW, RW)],
                    hist.at[pl.ds(c * NP + s * RW, RW)])


_deg = pl.kernel(
    _deg_body,
    out_type=jax.ShapeDtypeStruct((NC * NP, WC), jnp.float32),
    mesh=plsc.VectorSubcoreMesh(core_axis_name="c", subcore_axis_name="s",
                                num_cores=NC, num_subcores=NS),
    scratch_types=[
        pltpu.VMEM_SHARED((NP, WC), jnp.float32),
        pltpu.VMEM((KBLK,), jnp.int32),
        pltpu.VMEM((KBLK, WC), jnp.float32),
    ],
)


NBLK = EPT // KBLK
GB = 16


def _spmm_body(gtab, src2h, dst2h, out, acc, srcoff, dst2,
               msga, msgb, gsa, gsb, ssa, ssb):
    c = lax.axis_index("c")
    s = lax.axis_index("s")
    row0 = s * RW
    for r in range(2):
        q = 2 * c + r
        base = q * NP
        pltpu.sync_copy(gtab.at[pl.ds(base + row0, RW)],
                        acc.at[pl.ds(row0, RW)])
        plsc.subcore_barrier()

        def group(gi, carry):
            g0 = s * NBLK + gi * GB
            pltpu.sync_copy(src2h.at[pl.ds(g0, GB)], srcoff)
            pltpu.sync_copy(dst2h.at[pl.ds(g0, GB)], dst2)
            for i in range(GB):
                for j in range(KBLK // 16):
                    srcoff[i, pl.ds(16 * j, 16)] = (
                        srcoff[i, pl.ds(16 * j, 16)] + base)
            pltpu.async_copy(gtab.at[srcoff.at[0]], msga, gsa)
            for k in range(GB):
                cur, nxt = (msga, msgb) if k % 2 == 0 else (msgb, msga)
                csem, nsem = (gsa, gsb) if k % 2 == 0 else (gsb, gsa)
                css, nss = (ssa, ssb) if k % 2 == 0 else (ssb, ssa)
                if k + 1 < GB:
                    if k >= 1:
                        pltpu.make_async_copy(
                            nxt, acc.at[dst2.at[k - 1]], nss).wait()
                    pltpu.async_copy(gtab.at[srcoff.at[k + 1]], nxt, nsem)
                pltpu.make_async_copy(gtab.at[srcoff.at[k]], cur, csem).wait()
                pltpu.async_copy(cur, acc.at[dst2.at[k]], css, add=True)
            pltpu.make_async_copy(
                msga if (GB - 2) % 2 == 0 else msgb,
                acc.at[dst2.at[GB - 2]],
                ssa if (GB - 2) % 2 == 0 else ssb).wait()
            pltpu.make_async_copy(
                msga if (GB - 1) % 2 == 0 else msgb,
                acc.at[dst2.at[GB - 1]],
                ssa if (GB - 1) % 2 == 0 else ssb).wait()
            return carry

        lax.fori_loop(0, NBLK // GB, group, 0)
        plsc.subcore_barrier()
        pltpu.sync_copy(acc.at[pl.ds(row0, RW)],
                        out.at[pl.ds(q * NP + row0, RW)])


_spmm = pl.kernel(
    _spmm_body,
    out_type=jax.ShapeDtypeStruct((C * NP, WC), jnp.float32),
    mesh=plsc.VectorSubcoreMesh(core_axis_name="c", subcore_axis_name="s",
                                num_cores=NC, num_subcores=NS),
    scratch_types=[
        pltpu.VMEM_SHARED((NP, WC), jnp.float32),
        pltpu.VMEM((GB, KBLK), jnp.int32),
        pltpu.VMEM((GB, KBLK), jnp.int32),
        pltpu.VMEM((KBLK, WC), jnp.float32),
        pltpu.VMEM((KBLK, WC), jnp.float32),
        pltpu.SemaphoreType.DMA,
        pltpu.SemaphoreType.DMA,
        pltpu.SemaphoreType.DMA,
        pltpu.SemaphoreType.DMA,
    ],
)


def _dinv_col(hist_ref):
    hist = hist_ref[...]
    return lax.rsqrt(1.0 + hist[0] + hist[1])[:, 0:1]


def _tca_body(x_ref, hist_ref, w1_ref, sc1_ref, b1_ref, w1g_ref, out_ref):
    dcol = _dinv_col(hist_ref)
    x = x_ref[...]
    parts = [[] for _ in range(C)]
    for t in range(T):
        h = None
        for j in range(7):
            tt = t + j - 3
            if 0 <= tt < T:
                term = x[:, tt:tt + 1] * w1_ref[j:j + 1, :]
                h = term if h is None else h + term
        h = jnp.maximum(h * sc1_ref[...] + b1_ref[...], 0.0)
        m = jnp.dot(h, w1g_ref[...], preferred_element_type=jnp.float32)
        parts[t // 4].append(m * dcol)
    for q in range(C):
        out_ref[q] = jnp.concatenate(parts[q], axis=1)


_tca = pl.pallas_call(
    _tca_body,
    grid=(N // NB,),
    in_specs=[
        pl.BlockSpec((NB, T), lambda i: (i, 0)),
        pl.BlockSpec((2, NB, WC), lambda i: (0, i, 0)),
        pl.BlockSpec((7, K0), lambda i: (0, 0)),
        pl.BlockSpec((1, K0), lambda i: (0, 0)),
        pl.BlockSpec((1, K0), lambda i: (0, 0)),
        pl.BlockSpec((K0, D), lambda i: (0, 0)),
    ],
    out_specs=pl.BlockSpec((C, NB, WC), lambda i: (0, i, 0)),
    out_shape=jax.ShapeDtypeStruct((C, NP, WC), jnp.float32),
)


def _tcb_body(u_ref, hist_ref, b1t_ref, w2_ref, out_ref):
    dcol = _dinv_col(hist_ref)
    for q in range(C):
        z = jnp.maximum(u_ref[q] * dcol + b1t_ref[...], 0.0)
        ms = [jnp.dot(z[:, 32 * r:32 * r + 32], w2_ref[...],
                      preferred_element_type=jnp.float32) for r in range(4)]
        out_ref[q] = jnp.concatenate(ms, axis=1) * dcol


_tcb = pl.pallas_call(
    _tcb_body,
    grid=(N // NB,),
    in_specs=[
        pl.BlockSpec((C, NB, WC), lambda i: (0, i, 0)),
        pl.BlockSpec((2, NB, WC), lambda i: (0, i, 0)),
        pl.BlockSpec((1, WC), lambda i: (0, 0)),
        pl.BlockSpec((D, D), lambda i: (0, 0)),
    ],
    out_specs=pl.BlockSpec((C, NB, WC), lambda i: (0, i, 0)),
    out_shape=jax.ShapeDtypeStruct((C, NP, WC), jnp.float32),
)


def _tcc_body(u_ref, hist_ref, b2t_ref, w2c_ref, sc2_ref, bb2_ref,
              batch_ref, out_ref):
    i = pl.program_id(0)
    dcol = _dinv_col(hist_ref)
    g = jnp.concatenate(
        [jnp.maximum(u_ref[q] * dcol + b2t_ref[...], 0.0) for q in range(C)],
        axis=1)
    vs = []
    for t in range(T - 4):
        a = None
        for j in range(5):
            col = 32 * (t + j)
            term = jnp.dot(g[:, col:col + 32], w2c_ref[32 * j:32 * j + 32, :],
                           preferred_element_type=jnp.float32)
            a = term if a is None else a + term
        vs.append(a * sc2_ref[...] + bb2_ref[...])
    us = [jnp.maximum((vs[2 * t] + vs[2 * t + 1]) * 0.5, 0.0)
          for t in range(6)]
    onecol = (lax.broadcasted_iota(jnp.int32, (NB, 64), 1) == 0)
    uall = jnp.concatenate(us + [onecol.astype(jnp.float32)], axis=1)
    oh = (lax.broadcasted_iota(jnp.int32, (BP, NB), 0) == batch_ref[0])
    pp = jnp.dot(oh.astype(jnp.float32), uall,
                 preferred_element_type=jnp.float32)

    @pl.when(i == 0)
    def _():
        out_ref[...] = pp

    @pl.when(i > 0)
    def _():
        out_ref[...] += pp


_tcc = pl.pallas_call(
    _tcc_body,
    grid=(N // NB,),
    in_specs=[
        pl.BlockSpec((C, NB, WC), lambda i: (0, i, 0)),
        pl.BlockSpec((2, NB, WC), lambda i: (0, i, 0)),
        pl.BlockSpec((1, WC), lambda i: (0, 0)),
        pl.BlockSpec((5 * D, D), lambda i: (0, 0)),
        pl.BlockSpec((1, D), lambda i: (0, 0)),
        pl.BlockSpec((1, D), lambda i: (0, 0)),
        pl.BlockSpec((1, 1, NB), lambda i: (i, 0, 0)),
    ],
    out_specs=pl.BlockSpec((BP, 256), lambda i: (0, 0)),
    out_shape=jax.ShapeDtypeStruct((BP, 256), jnp.float32),
)


def _tcd_body(p_ref, w3_ref, sc3_ref, b3_ref, lw_ref, lb_ref, out_ref):
    pe = p_ref[...]
    cnt = jnp.maximum(pe[:, 192:193], 1.0)
    p = pe[:, 0:192] / cnt
    vs = []
    for t in range(5):
        a = (jnp.dot(p[:, 32 * t:32 * t + 32], w3_ref[0:32, :],
                     preferred_element_type=jnp.float32) +
             jnp.dot(p[:, 32 * t + 32:32 * t + 64], w3_ref[32:64, :],
                     preferred_element_type=jnp.float32))
        vs.append(a * sc3_ref[...] + b3_ref[...])
    f0 = jnp.maximum((vs[0] + vs[1]) * 0.5, 0.0)
    f1 = jnp.maximum((vs[2] + vs[3]) * 0.5, 0.0)
    flat = jnp.concatenate([f0, f1], axis=1)
    logits = jnp.dot(flat, lw_ref[...],
                     preferred_element_type=jnp.float32) + lb_ref[...]
    mx = jnp.max(logits, axis=1, keepdims=True)
    e = jnp.exp(logits - mx)
    lse = jnp.log(jnp.sum(e, axis=1, keepdims=True)) + mx
    out_ref[...] = logits - lse


_tcd = pl.pallas_call(
    _tcd_body,
    in_specs=[
        pl.BlockSpec((BP, 256), lambda: (0, 0)),
        pl.BlockSpec((64, D), lambda: (0, 0)),
        pl.BlockSpec((1, D), lambda: (0, 0)),
        pl.BlockSpec((1, D), lambda: (0, 0)),
        pl.BlockSpec((64, 128), lambda: (0, 0)),
        pl.BlockSpec((1, 128), lambda: (0, 0)),
    ],
    out_specs=pl.BlockSpec((BP, 128), lambda: (0, 0)),
    out_shape=jax.ShapeDtypeStruct((BP, 128), jnp.float32),
)


def kernel(x, edge_index, batch, conv1_w, bn1_g, bn1_b, gcn1_w, gcn1_b,
           gcn2_w, gcn2_b, conv2_w, bn2_g, bn2_b, conv3_w, bn3_g, bn3_b,
           lin_w, lin_b):
    f32 = jnp.float32
    pad = ET - E
    srcp = jnp.concatenate([edge_index[0],
                            jnp.zeros((pad,), jnp.int32)])
    dstp = jnp.concatenate([edge_index[1],
                            jnp.full((pad,), N, jnp.int32)])
    ones_h = jnp.ones((KBLK, WC), f32)
    zeros_h = jnp.zeros((RW, WC), f32)
    hist = _deg(dstp, ones_h, zeros_h).reshape(2, NP, WC)

    rs = 1.0 / jnp.sqrt(jnp.asarray(1.0 + 1e-5, f32))
    w1 = conv1_w[:, 0, 0, :]
    sc1 = (bn1_g * rs).reshape(1, K0)
    b1 = bn1_b.reshape(1, K0)
    g1 = _tca(x, hist, w1, sc1, b1, gcn1_w)

    def _jnp_spmm(gtab, srcp, dstp):
        out = gtab
        for q in range(C):
            seg = gtab[q * NP:(q + 1) * NP]
            acc = seg.at[dstp].add(seg[srcp])
            out = out.at[q * NP:(q + 1) * NP].set(acc)
        return out

    src2h = srcp.reshape(ET // KBLK, KBLK)
    dst2h = dstp.reshape(ET // KBLK, KBLK)
    u1 = _spmm(g1.reshape(C * NP, WC), src2h, dst2h).reshape(C, NP, WC)
    b1t = jnp.tile(gcn1_b, C).reshape(1, WC)
    g2 = _tcb(u1, hist, b1t, gcn2_w)

    u2 = _spmm(g2.reshape(C * NP, WC), src2h, dst2h).reshape(C, NP, WC)
    b2t = jnp.tile(gcn2_b, C).reshape(1, WC)
    w2c = conv2_w[:, 0, :, :].reshape(5 * D, D)
    sc2 = (bn2_g * rs).reshape(1, D)
    bb2 = bn2_b.reshape(1, D)
    pext = _tcc(u2, hist, b2t, w2c, sc2, bb2, batch.reshape(N // NB, 1, NB))

    w3 = conv3_w[:, 0, :, :].reshape(2 * D, D)
    sc3 = (bn3_g * rs).reshape(1, D)
    b3 = bn3_b.reshape(1, D)
    lw = jnp.concatenate([lin_w, jnp.zeros((2 * D, 124), f32)], axis=1)
    lb = jnp.concatenate([lin_b, jnp.full((124,), -1e30, f32)]).reshape(1, 128)
    out = _tcd(pext, w3, sc3, b3, lw, lb)
    return out[:B, :4]
